# baseline (speedup 1.0000x reference)
import functools

import jax
import jax.numpy as jnp
from jax import lax
from jax.experimental import pallas as pl
from jax.experimental.pallas import tpu as pltpu
from jax.experimental.pallas import tpu_sc as plsc

B = 4096
F = 26
V = 100000
D = 16
ND = 13

VC = 100000
VB = 102400
VPAD = 323584

NC = 2
NS = 16
L = 16
NW = NC * NS
BPW = B // NW
EPW = BPW * F
CHUNK = 128
NCHUNK = EPW // CHUNK
NG = BPW // L

_mesh = plsc.VectorSubcoreMesh(core_axis_name="c", subcore_axis_name="s")


def _colsum_body(t_ref, d_ref, w_ref, cs_ref, sq_ref, lin_ref):
    x = t_ref[0]
    ones = jnp.ones((1, D), jnp.float32)
    dn = (((1,), (0,)), ((), ()))
    cs = jax.lax.dot_general(ones, x, dn, preferred_element_type=jnp.float32)
    sq = jax.lax.dot_general(ones, x * x, dn,
                             preferred_element_type=jnp.float32)
    cs_ref[pl.ds(0, VC)] = cs[0]
    sq_ref[pl.ds(0, VC)] = sq[0]

    @pl.when(pl.program_id(0) == 0)
    def _():
        w = w_ref[...]
        w13 = jax.lax.slice(w, (0,), (ND,)).reshape(ND, 1)
        bias = jax.lax.slice(w, (ND,), (ND + 1,))
        lin = jax.lax.dot_general(
            d_ref[...], w13, (((1,), (0,)), ((), ())),
            preferred_element_type=jnp.float32)
        lin_ref[...] = lin[:, 0] + bias[0]


_colsum_tc = pl.pallas_call(
    _colsum_body,
    grid=(F,),
    in_specs=[pl.BlockSpec((1, D, VC), lambda f: (f, 0, 0)),
              pl.BlockSpec((B, ND), lambda f: (0, 0)),
              pl.BlockSpec((ND + 1,), lambda f: (0,))],
    out_specs=[pl.BlockSpec((VB,), lambda f: (f,)),
               pl.BlockSpec((VB,), lambda f: (f,)),
               pl.BlockSpec((B,), lambda f: (0,))],
    out_shape=[jax.ShapeDtypeStruct((F * VPAD,), jnp.float32),
               jax.ShapeDtypeStruct((F * VPAD,), jnp.float32),
               jax.ShapeDtypeStruct((B,), jnp.float32)],
)


_SCRATCH = [
    pltpu.VMEM((F, CHUNK), jnp.int32),
    pltpu.VMEM((F, CHUNK), jnp.float32),
    pltpu.VMEM((F, CHUNK), jnp.float32),
    pltpu.VMEM((BPW,), jnp.float32),
    pltpu.VMEM((BPW,), jnp.float32),
    pltpu.SemaphoreType.DMA,
]


def _fm_body(cs, sq, sparse3, lin, out, idx_v, cs_v, sq_v, lin_v, out_v,
             sem):
    wid = lax.axis_index("s") * NC + lax.axis_index("c")

    pltpu.sync_copy(sparse3.at[:, wid], idx_v)
    pltpu.sync_copy(lin.at[pl.ds(wid * BPW, BPW)], lin_v)

    for f in range(F):
        for j in range(CHUNK // L):
            sl = pl.ds(j * L, L)
            idx_v[f, sl] = idx_v[f, sl] + (f * VB)

    handles = [
        pltpu.async_copy(cs.at[idx_v.at[f]], cs_v.at[f], sem)
        for f in range(F)
    ] + [
        pltpu.async_copy(sq.at[idx_v.at[f]], sq_v.at[f], sem)
        for f in range(F)
    ]
    for h in handles:
        h.wait()

    def group(g, carry):
        sl = pl.ds(g * L, L)
        sv = cs_v[0, sl]
        qv = sq_v[0, sl]
        for f in range(1, F):
            sv = sv + cs_v[f, sl]
            qv = qv + sq_v[f, sl]
        z = lin_v[sl] + sv + 0.5 * (sv * sv - qv)
        out_v[sl] = 1.0 / (1.0 + jnp.exp(-z))
        return carry

    lax.fori_loop(0, NG, group, 0)

    pltpu.sync_copy(out_v, out.at[pl.ds(wid * BPW, BPW)])


_fm_sc = pl.kernel(
    _fm_body,
    mesh=_mesh,
    compiler_params=pltpu.CompilerParams(
        needs_layout_passes=False, use_tc_tiling_on_sc=False),
    out_type=jax.ShapeDtypeStruct((B,), jnp.float32),
    scratch_types=_SCRATCH,
)


def kernel(dense_input, sparse_input, emb_tables, W_lin, b_lin):
    tables_dv = jnp.transpose(emb_tables, (0, 2, 1))
    wb = jnp.concatenate([W_lin.reshape(-1), b_lin.reshape(-1)])
    cs, sq, lin = _colsum_tc(tables_dv, dense_input, wb)

    sparse3 = sparse_input.astype(jnp.int32).T.reshape(F, NW, CHUNK)
    out = _fm_sc(cs, sq, sparse3, lin)
    return out.reshape(B, 1)


# speedup vs baseline: 1.0268x; 1.0268x over previous
import functools

import jax
import jax.numpy as jnp
from jax import lax
from jax.experimental import pallas as pl
from jax.experimental.pallas import tpu as pltpu
from jax.experimental.pallas import tpu_sc as plsc

B = 4096
F = 26
V = 100000
D = 16
ND = 13

VC = 100000
VB = 102400
VPAD = 323584

NC = 2
NS = 16
L = 16
NW = NC * NS
BPW = B // NW
EPW = BPW * F
CHUNK = 128
NCHUNK = EPW // CHUNK
NG = BPW // L

_mesh = plsc.VectorSubcoreMesh(core_axis_name="c", subcore_axis_name="s")


def _colsum_body(t_ref, cs_ref, sq_ref):
    x = t_ref[0]
    ones = jnp.ones((1, D), jnp.float32)
    dn = (((1,), (0,)), ((), ()))
    cs = jax.lax.dot_general(ones, x, dn, preferred_element_type=jnp.float32)
    sq = jax.lax.dot_general(ones, x * x, dn,
                             preferred_element_type=jnp.float32)
    cs_ref[pl.ds(0, VC)] = cs[0]
    sq_ref[pl.ds(0, VC)] = sq[0]


_colsum_tc = pl.pallas_call(
    _colsum_body,
    grid=(F,),
    in_specs=[pl.BlockSpec((1, D, VC), lambda f: (f, 0, 0))],
    out_specs=[pl.BlockSpec((VB,), lambda f: (f,)),
               pl.BlockSpec((VB,), lambda f: (f,))],
    out_shape=[jax.ShapeDtypeStruct((F * VPAD,), jnp.float32),
               jax.ShapeDtypeStruct((F * VPAD,), jnp.float32)],
)


_SCRATCH = [
    pltpu.VMEM((F, CHUNK), jnp.int32),
    pltpu.VMEM((F, CHUNK), jnp.float32),
    pltpu.VMEM((F, CHUNK), jnp.float32),
    pltpu.VMEM((ND, CHUNK), jnp.float32),
    pltpu.VMEM((ND + 1, L), jnp.float32),
    pltpu.VMEM((BPW,), jnp.float32),
    pltpu.SemaphoreType.DMA,
]


def _fm_body(cs, sq, sparse3, dense3, wb, out, idx_v, cs_v, sq_v, dense_v,
             wb_v, out_v, sem):
    wid = lax.axis_index("s") * NC + lax.axis_index("c")

    pltpu.sync_copy(sparse3.at[:, wid], idx_v)
    h_dense = pltpu.async_copy(dense3.at[:, wid], dense_v, sem)
    h_wb = pltpu.async_copy(wb, wb_v, sem)

    handles = []
    for f in range(F):
        for j in range(CHUNK // L):
            sl = pl.ds(j * L, L)
            idx_v[f, sl] = idx_v[f, sl] + (f * VB)
        handles.append(pltpu.async_copy(cs.at[idx_v.at[f]], cs_v.at[f], sem))
        handles.append(pltpu.async_copy(sq.at[idx_v.at[f]], sq_v.at[f], sem))
    h_dense.wait()
    h_wb.wait()
    for h in handles:
        h.wait()

    ws = [wb_v[j, :] for j in range(ND)]
    bias = wb_v[ND, :]

    for g in range(NG):
        sl = pl.ds(g * L, L)
        sv = cs_v[0, sl]
        qv = sq_v[0, sl]
        for f in range(1, F):
            sv = sv + cs_v[f, sl]
            qv = qv + sq_v[f, sl]
        lin = dense_v[0, sl] * ws[0] + bias
        for j in range(1, ND):
            lin = lin + dense_v[j, sl] * ws[j]
        z = lin + sv + 0.5 * (sv * sv - qv)
        out_v[sl] = 1.0 / (1.0 + jnp.exp(-z))

    pltpu.sync_copy(out_v, out.at[pl.ds(wid * BPW, BPW)])


_fm_sc = pl.kernel(
    _fm_body,
    mesh=_mesh,
    compiler_params=pltpu.CompilerParams(
        needs_layout_passes=False, use_tc_tiling_on_sc=False),
    out_type=jax.ShapeDtypeStruct((B,), jnp.float32),
    scratch_types=_SCRATCH,
)


def kernel(dense_input, sparse_input, emb_tables, W_lin, b_lin):
    tables_dv = jnp.transpose(emb_tables, (0, 2, 1))
    cs, sq = _colsum_tc(tables_dv)

    sparse3 = sparse_input.astype(jnp.int32).T.reshape(F, NW, CHUNK)
    dense3 = dense_input.T.reshape(ND, NW, BPW)
    wb = jnp.broadcast_to(
        jnp.concatenate([W_lin.reshape(-1), b_lin.reshape(-1)])[:, None],
        (ND + 1, L))
    out = _fm_sc(cs, sq, sparse3, dense3, wb)
    return out.reshape(B, 1)
